# baseline (speedup 1.0000x reference)
import functools

import jax
import jax.numpy as jnp
from jax import lax
from jax.experimental import pallas as pl
from jax.experimental.pallas import tpu as pltpu
from jax.experimental.pallas import tpu_sc as plsc

D = 768
S = 8192
B = 4
NROWS = B * S
NW = 32
ROWS_PER_W = NROWS // NW
SPANS_PER_BATCH = S // ROWS_PER_W
R = 32
NCHUNK = ROWS_PER_W // R
LG = D // 16


def _sc_call(x2d, st_rep, pe):
    mesh = plsc.VectorSubcoreMesh(core_axis_name="c", subcore_axis_name="s")

    @functools.partial(
        pl.kernel,
        mesh=mesh,
        out_type=jax.ShapeDtypeStruct((NROWS, D), jnp.float32),
        scratch_types=[
            pltpu.VMEM((R, D), jnp.float32),
            pltpu.VMEM((R, D), jnp.float32),
            pltpu.VMEM((R,), jnp.int32),
            pltpu.VMEM((16,), jnp.int32),
            pltpu.SemaphoreType.DMA,
            pltpu.SemaphoreType.DMA,
        ],
    )
    def k(x_hbm, st_hbm, pe_hbm, out_hbm, xb, peb, idxb, stv, semg, semx):
        wid = lax.axis_index("c") * 16 + lax.axis_index("s")
        batch = wid // SPANS_PER_BATCH
        s_base = (wid % SPANS_PER_BATCH) * ROWS_PER_W

        pltpu.sync_copy(st_hbm.at[wid], stv)
        lanes = lax.iota(jnp.int32, 16)
        st = stv[...]

        def chunk(g, carry):
            s0 = s_base + g * R
            row0 = batch * S + s0
            for j in range(R // 16):
                idxb[pl.ds(j * 16, 16)] = jnp.maximum(s0 + j * 16 + lanes - st, 0)
            cx = pltpu.async_copy(x_hbm.at[pl.ds(row0, R)], xb, semx)
            cg = pltpu.async_copy(pe_hbm.at[idxb], peb, semg)
            cx.wait()
            cg.wait()

            def row(i, c2):
                for j in range(LG):
                    plsc.addupdate(xb.at[i, pl.ds(j * 16, 16)],
                                   peb[i, pl.ds(j * 16, 16)])
                return c2

            lax.fori_loop(0, R, row, 0)
            pltpu.sync_copy(xb, out_hbm.at[pl.ds(row0, R)])
            return carry

        lax.fori_loop(0, NCHUNK, chunk, 0)

    return k(x2d, st_rep, pe)


def kernel(x, stidx, pe):
    x2d = x.reshape(NROWS, D)
    st_rep = jnp.repeat(stidx.astype(jnp.int32),
                        SPANS_PER_BATCH * 16).reshape(NW, 16)
    out = _sc_call(x2d, st_rep, pe)
    return out.reshape(B, S, D)
